# baseline (speedup 1.0000x reference)
import functools

import jax
import jax.numpy as jnp
from jax import lax
from jax.experimental import pallas as pl
from jax.experimental.pallas import tpu as pltpu
from jax.experimental.pallas import tpu_sc as plsc

_B = 16384
_NC = 2
_NS = 16
_NW = _NC * _NS
_BPW = _B // _NW
_CHUNK = 128

_BLK = 1024
_GRID = _B // _BLK


_ROWS_PER_CHUNK = _CHUNK // 2
_NCHUNK = _BPW // _ROWS_PER_CHUNK


def _sc_gather_body(table_hbm, idx_hbm, out_hbm, idx_v, rows_v, sem):
    wid = lax.axis_index("s") * _NC + lax.axis_index("c")
    base = wid * _BPW
    pltpu.sync_copy(idx_hbm.at[pl.ds(wid * _NCHUNK, _NCHUNK)], idx_v)
    copies = []
    for j in range(_NCHUNK):
        copies.append(
            pltpu.async_copy(
                table_hbm.at[idx_v.at[j]],
                rows_v.at[pl.ds(j * _CHUNK, _CHUNK)],
                sem,
            )
        )
    for c in copies:
        c.wait()
    for j in range(_NCHUNK):
        pltpu.sync_copy(
            rows_v.at[pl.ds(j * _CHUNK, _ROWS_PER_CHUNK)],
            out_hbm.at[pl.ds(base + j * _ROWS_PER_CHUNK, _ROWS_PER_CHUNK)],
        )


@functools.partial(jax.jit, static_argnums=())
def _sc_gather(emb_table, domain_idx):
    mesh = plsc.VectorSubcoreMesh(core_axis_name="c", subcore_axis_name="s")
    doubled = domain_idx.astype(jnp.int32) * 2
    spread = jnp.stack([doubled, jnp.zeros_like(doubled)], axis=-1)
    idx2d = spread.reshape(_NW * _NCHUNK, _CHUNK)
    return pl.kernel(
        _sc_gather_body,
        out_type=jax.ShapeDtypeStruct((_B, 4), jnp.float32),
        mesh=mesh,
        scratch_types=[
            pltpu.VMEM((_NCHUNK, _CHUNK), jnp.int32),
            pltpu.VMEM((_NCHUNK * _CHUNK, 4), jnp.float32),
            pltpu.SemaphoreType.DMA,
        ],
        compiler_params=pltpu.CompilerParams(use_tc_tiling_on_sc=False),
    )(emb_table, idx2d)


def _tc_mlp_body(f_ref, e_ref, w1a_ref, w1b_ref, b1_ref, w2_ref, b2_ref, o_ref):
    h = jnp.dot(f_ref[...], w1a_ref[...], preferred_element_type=jnp.float32)
    h = h + jnp.dot(e_ref[...], w1b_ref[...], preferred_element_type=jnp.float32)
    h = jnp.maximum(h + b1_ref[...], 0.0)
    o_ref[...] = (
        jnp.dot(h, w2_ref[...], preferred_element_type=jnp.float32) + b2_ref[0, 0]
    )


def _tc_mlp(features, dom_emb, W1a, W1b, b1, W2, b2):
    return pl.pallas_call(
        _tc_mlp_body,
        grid=(_GRID,),
        in_specs=[
            pl.BlockSpec((_BLK, 128), lambda i: (i, 0)),
            pl.BlockSpec((_BLK, 4), lambda i: (i, 0)),
            pl.BlockSpec((128, 64), lambda i: (0, 0)),
            pl.BlockSpec((4, 64), lambda i: (0, 0)),
            pl.BlockSpec((1, 64), lambda i: (0, 0)),
            pl.BlockSpec((64, 1), lambda i: (0, 0)),
            pl.BlockSpec((1, 1), lambda i: (0, 0)),
        ],
        out_specs=pl.BlockSpec((_BLK, 1), lambda i: (i, 0)),
        out_shape=jax.ShapeDtypeStruct((_B, 1), jnp.float32),
    )(features, dom_emb, W1a, W1b, b1, W2, b2)


def kernel(features, domain_idx, emb_table, W1, b1, W2, b2):
    dom_emb = _sc_gather(emb_table, domain_idx)
    W1a = W1[:128]
    W1b = W1[128:]
    out = _tc_mlp(
        features,
        dom_emb,
        W1a,
        W1b,
        b1.reshape(1, 64),
        W2,
        b2.reshape(1, 1),
    )
    return out.reshape(_B)


# speedup vs baseline: 1.2345x; 1.2345x over previous
import functools

import jax
import jax.numpy as jnp
from jax import lax
from jax.experimental import pallas as pl
from jax.experimental.pallas import tpu as pltpu
from jax.experimental.pallas import tpu_sc as plsc

_B = 16384
_V2 = 31250
_NC = 2
_NS = 16
_NW = _NC * _NS
_BPW = _B // _NW
_CHUNK = 128
_NCHUNK = _BPW // _CHUNK

_BLK = 1024
_GRID = _B // _BLK


def _sc_gather_body(table_hbm, gidx_hbm, col_hbm, out_hbm, gidx_v, col_v, rows_v, emb_v, sem):
    wid = lax.axis_index("s") * _NC + lax.axis_index("c")
    base = wid * _BPW
    pltpu.sync_copy(gidx_hbm.at[pl.ds(wid * _NCHUNK, _NCHUNK)], gidx_v)
    pltpu.sync_copy(col_hbm.at[wid], col_v)
    copies = []
    for j in range(_NCHUNK):
        copies.append(
            pltpu.async_copy(
                table_hbm.at[gidx_v.at[j]],
                rows_v.at[pl.ds(j * _CHUNK, _CHUNK)],
                sem,
            )
        )
    for c in copies:
        c.wait()
    for k in range(_BPW // 16):
        r0 = k * 16
        row_ids = r0 + lax.iota(jnp.int32, 16)
        col0 = col_v[pl.ds(r0, 16)]
        for j in range(4):
            vals = plsc.load_gather(rows_v, [row_ids, col0 + j])
            emb_v[j, pl.ds(r0, 16)] = vals
    pltpu.sync_copy(emb_v, out_hbm.at[:, pl.ds(base, _BPW)])


@jax.jit
def _sc_gather(table2, gidx2d, col2d):
    mesh = plsc.VectorSubcoreMesh(core_axis_name="c", subcore_axis_name="s")
    return pl.kernel(
        _sc_gather_body,
        out_type=jax.ShapeDtypeStruct((4, _B), jnp.float32),
        mesh=mesh,
        scratch_types=[
            pltpu.VMEM((_NCHUNK, _CHUNK), jnp.int32),
            pltpu.VMEM((_BPW,), jnp.int32),
            pltpu.VMEM((_BPW, 128), jnp.float32),
            pltpu.VMEM((4, _BPW), jnp.float32),
            pltpu.SemaphoreType.DMA,
        ],
        compiler_params=pltpu.CompilerParams(needs_layout_passes=False),
    )(table2, gidx2d, col2d)


def _tc_mlp_body(f_ref, e_ref, w1a_ref, w1b_ref, b1_ref, w2_ref, b2_ref, o_ref):
    h = jnp.dot(f_ref[...], w1a_ref[...], preferred_element_type=jnp.float32)
    h = h + lax.dot_general(
        e_ref[...], w1b_ref[...], (((0,), (0,)), ((), ())),
        preferred_element_type=jnp.float32,
    )
    h = jnp.maximum(h + b1_ref[...], 0.0)
    o_ref[...] = (
        jnp.dot(h, w2_ref[...], preferred_element_type=jnp.float32) + b2_ref[0, 0]
    )


def _tc_mlp(features, dom_emb_t, W1a, W1b, b1, W2, b2):
    return pl.pallas_call(
        _tc_mlp_body,
        grid=(_GRID,),
        in_specs=[
            pl.BlockSpec((_BLK, 128), lambda i: (i, 0)),
            pl.BlockSpec((4, _BLK), lambda i: (0, i)),
            pl.BlockSpec((128, 64), lambda i: (0, 0)),
            pl.BlockSpec((4, 64), lambda i: (0, 0)),
            pl.BlockSpec((1, 64), lambda i: (0, 0)),
            pl.BlockSpec((64, 1), lambda i: (0, 0)),
            pl.BlockSpec((1, 1), lambda i: (0, 0)),
        ],
        out_specs=pl.BlockSpec((_BLK, 1), lambda i: (i, 0)),
        out_shape=jax.ShapeDtypeStruct((_B, 1), jnp.float32),
    )(features, dom_emb_t, W1a, W1b, b1, W2, b2)


def kernel(features, domain_idx, emb_table, W1, b1, W2, b2):
    table2 = emb_table.reshape(_V2, 128)
    idx = domain_idx.astype(jnp.int32)
    gidx2d = (idx >> 5).reshape(_NW * _NCHUNK, _CHUNK)
    col2d = ((idx & 31) << 2).reshape(_NW, _BPW)
    dom_emb_t = _sc_gather(table2, gidx2d, col2d)
    out = _tc_mlp(
        features,
        dom_emb_t,
        W1[:128],
        W1[128:],
        b1.reshape(1, 64),
        W2,
        b2.reshape(1, 1),
    )
    return out.reshape(_B)
